# baseline (speedup 1.0000x reference)
import functools

import jax
import jax.numpy as jnp
from jax import lax
from jax.experimental import pallas as pl
from jax.experimental.pallas import tpu as pltpu
from jax.experimental.pallas import tpu_sc as plsc

N = 10000
E = 320000
D = 128
H = 128
G = 64
OUT = 10

NPAD = 10240
RPT = NPAD // 16
CH = 128
CHUNKS = 2560
E_PAD = CHUNKS * CH
CPW = CHUNKS // 32
SLAB = 40
BN = 512
NB = NPAD // BN

_mesh = plsc.VectorSubcoreMesh(core_axis_name="c", subcore_axis_name="s")


def _sc_deg_body(dst_hbm, ones_hbm, out_hbm, dstv, onesv, acc, ss0, ss1):
    c = lax.axis_index("c")
    s = lax.axis_index("s")
    r0 = s * RPT
    pltpu.sync_copy(ones_hbm, onesv)

    @pl.loop(0, RPT // CH)
    def _(k):
        pltpu.sync_copy(onesv, acc.at[pl.ds(r0 + k * CH, CH)])

    base = (c * 16 + s) * CPW
    pltpu.sync_copy(dst_hbm.at[pl.ds(base, CPW)], dstv)
    plsc.subcore_barrier()

    pltpu.async_copy(onesv, acc.at[dstv.at[0]], ss0, add=True)
    pltpu.async_copy(onesv, acc.at[dstv.at[1]], ss1, add=True)

    @pl.loop(0, CPW, step=2)
    def _(j):
        pltpu.make_async_copy(onesv, acc.at[dstv.at[j]], ss0).wait()

        @pl.when(j + 2 < CPW)
        def _():
            pltpu.async_copy(onesv, acc.at[dstv.at[j + 2]], ss0, add=True)

        pltpu.make_async_copy(onesv, acc.at[dstv.at[j + 1]], ss1).wait()

        @pl.when(j + 3 < CPW)
        def _():
            pltpu.async_copy(onesv, acc.at[dstv.at[j + 3]], ss1, add=True)

    plsc.subcore_barrier()
    pltpu.sync_copy(acc.at[pl.ds(r0, RPT)], out_hbm.at[c, pl.ds(r0, RPT)])


def _sc_agg_body(src_hbm, dst_hbm, m_hbm, out_hbm, srcv, dstv, rows0, rows1,
                 acc, gs0, gs1, ss0, ss1):
    c = lax.axis_index("c")
    s = lax.axis_index("s")
    r0 = s * RPT
    pltpu.sync_copy(m_hbm.at[pl.ds(r0, RPT)], acc.at[pl.ds(r0, RPT)])
    base = (c * 16 + s) * CPW
    plsc.subcore_barrier()

    for p in range(CPW // SLAB):
        pltpu.sync_copy(src_hbm.at[pl.ds(base + p * SLAB, SLAB)], srcv)
        pltpu.sync_copy(dst_hbm.at[pl.ds(base + p * SLAB, SLAB)], dstv)
        pltpu.async_copy(m_hbm.at[srcv.at[0]], rows0, gs0)

        @pl.loop(0, SLAB, step=2)
        def _(j):
            pltpu.async_copy(m_hbm.at[srcv.at[j + 1]], rows1, gs1)
            pltpu.make_async_copy(m_hbm.at[srcv.at[j]], rows0, gs0).wait()
            pltpu.sync_copy(rows0, acc.at[dstv.at[j]], add=True)

            @pl.when(j + 2 < SLAB)
            def _():
                pltpu.async_copy(m_hbm.at[srcv.at[j + 2]], rows0, gs0)

            pltpu.make_async_copy(m_hbm.at[srcv.at[j + 1]], rows1, gs1).wait()
            pltpu.sync_copy(rows1, acc.at[dstv.at[j + 1]], add=True)

    plsc.subcore_barrier()
    pltpu.sync_copy(acc.at[pl.ds(r0, RPT)], out_hbm.at[c, pl.ds(r0, RPT)])


_sc_deg = pl.kernel(
    _sc_deg_body,
    out_type=jax.ShapeDtypeStruct((2, NPAD, H), jnp.float32),
    mesh=_mesh,
    scratch_types=[
        pltpu.VMEM((CPW, CH), jnp.int32),
        pltpu.VMEM((CH, H), jnp.float32),
        pltpu.VMEM_SHARED((NPAD, H), jnp.float32),
        pltpu.SemaphoreType.DMA,
        pltpu.SemaphoreType.DMA,
    ],
)

_sc_agg = pl.kernel(
    _sc_agg_body,
    out_type=jax.ShapeDtypeStruct((2, NPAD, H), jnp.float32),
    mesh=_mesh,
    scratch_types=[
        pltpu.VMEM((SLAB, CH), jnp.int32),
        pltpu.VMEM((SLAB, CH), jnp.int32),
        pltpu.VMEM((CH, H), jnp.float32),
        pltpu.VMEM((CH, H), jnp.float32),
        pltpu.VMEM_SHARED((NPAD, H), jnp.float32),
        pltpu.SemaphoreType.DMA,
        pltpu.SemaphoreType.DMA,
        pltpu.SemaphoreType.DMA,
        pltpu.SemaphoreType.DMA,
    ],
)


def _tc_mm_body(x, w, u_out):
    u_out[...] = jnp.dot(x[...], w[...], preferred_element_type=jnp.float32)


def _tc_pre_body(dega, degb, u, m_out, dinv_out):
    i = pl.program_id(0)
    deg = dega[:, 0:1] + degb[:, 0:1] - 1.0
    rows = i * BN + lax.broadcasted_iota(jnp.int32, (BN, 1), 0)
    dinv = jnp.where(rows < N, lax.rsqrt(jnp.maximum(deg, 1e-12)), 0.0)
    m_out[...] = u[...] * dinv
    dinv_out[...] = dinv


def _tc_mid_body(ta, tb, m, dinv, b, w, out):
    t = ta[...] + tb[...] - m[...]
    h = jnp.maximum(dinv[...] * t + b[...], 0.0)
    out[...] = jnp.dot(h, w[...], preferred_element_type=jnp.float32) * dinv[...]


def _tc_post_body(ta, tb, m, dinv, b, bt, wih_t, bih, bhh, wout_t, bout,
                  out, sums, cnt):
    i = pl.program_id(0)

    @pl.when(i == 0)
    def _():
        sums[...] = jnp.zeros((G, H), jnp.float32)
        cnt[...] = jnp.zeros((G, 1), jnp.float32)

    t = ta[...] + tb[...] - m[...]
    h = jnp.maximum(dinv[...] * t + b[...], 0.0)
    bvec = bt[...][0]
    gids = lax.broadcasted_iota(jnp.int32, (G, 1), 0)
    p = (bvec == gids).astype(jnp.float32)
    sums[...] += jnp.dot(p, h, preferred_element_type=jnp.float32)
    cnt[...] += jnp.sum(p, axis=1, keepdims=True)

    @pl.when(i == NB - 1)
    def _():
        pooled = sums[...] / jnp.maximum(cnt[...], 1.0)
        gates = (jnp.dot(pooled, wih_t[...], preferred_element_type=jnp.float32)
                 + bih[...] + bhh[...])
        ig = jax.nn.sigmoid(gates[:, 0:H])
        gg = jnp.tanh(gates[:, 2 * H:3 * H])
        og = jax.nn.sigmoid(gates[:, 3 * H:4 * H])
        hn = og * jnp.tanh(ig * gg)
        out[...] = (jnp.dot(hn, wout_t[...], preferred_element_type=jnp.float32)
                    + bout[...])


def _row_spec(width):
    return pl.BlockSpec((BN, width), lambda i: (i, 0))


def _full_spec(shape):
    return pl.BlockSpec(shape, lambda i: (0,) * len(shape))


_tc_mm = pl.pallas_call(
    _tc_mm_body,
    grid=(NB,),
    in_specs=[_row_spec(D), _full_spec((D, H))],
    out_specs=_row_spec(H),
    out_shape=jax.ShapeDtypeStruct((NPAD, H), jnp.float32),
)

_tc_pre = pl.pallas_call(
    _tc_pre_body,
    grid=(NB,),
    in_specs=[_row_spec(H), _row_spec(H), _row_spec(H)],
    out_specs=[_row_spec(H), _row_spec(1)],
    out_shape=[jax.ShapeDtypeStruct((NPAD, H), jnp.float32),
               jax.ShapeDtypeStruct((NPAD, 1), jnp.float32)],
)

_tc_mid = pl.pallas_call(
    _tc_mid_body,
    grid=(NB,),
    in_specs=[_row_spec(H), _row_spec(H), _row_spec(H), _row_spec(1),
              _full_spec((1, H)), _full_spec((H, H))],
    out_specs=_row_spec(H),
    out_shape=jax.ShapeDtypeStruct((NPAD, H), jnp.float32),
)

_tc_post = pl.pallas_call(
    _tc_post_body,
    grid=(NB,),
    in_specs=[_row_spec(H), _row_spec(H), _row_spec(H), _row_spec(1),
              _full_spec((1, H)),
              pl.BlockSpec((1, 1, BN), lambda i: (i, 0, 0)),
              _full_spec((H, 4 * H)), _full_spec((1, 4 * H)),
              _full_spec((1, 4 * H)), _full_spec((H, 16)),
              _full_spec((1, 16))],
    out_specs=_full_spec((G, 16)),
    out_shape=jax.ShapeDtypeStruct((G, 16), jnp.float32),
    scratch_shapes=[pltpu.VMEM((G, H), jnp.float32),
                    pltpu.VMEM((G, 1), jnp.float32)],
)


def kernel(x, edge_index, batch, W1, b1, W2, b2, W_ih, W_hh, b_ih, b_hh, W_out, b_out):
    src = edge_index[0]
    dst = edge_index[1]
    pad_e = E_PAD - E
    ar = jnp.arange(pad_e, dtype=jnp.int32)
    src2d = jnp.concatenate([src, ar % N]).reshape(CHUNKS, CH)
    dst2d = jnp.concatenate([dst, N + ar % (NPAD - N)]).reshape(CHUNKS, CH)
    x_pad = jnp.pad(x, ((0, NPAD - N), (0, 0)))
    batch3d = jnp.pad(batch, (0, NPAD - N), constant_values=G).reshape(NB, 1, BN)
    ones128 = jnp.ones((CH, H), jnp.float32)
    u1 = _tc_mm(x_pad, W1)
    degp = _sc_deg(dst2d, ones128)
    m1, dinv = _tc_pre(degp[0], degp[1], u1)
    t1 = _sc_agg(src2d, dst2d, m1)
    m2 = _tc_mid(t1[0], t1[1], m1, dinv, b1.reshape(1, H), W2)
    t2 = _sc_agg(src2d, dst2d, m2)
    wout_t = jnp.pad(W_out.T, ((0, 0), (0, 16 - OUT)))
    bout = jnp.pad(b_out, (0, 16 - OUT)).reshape(1, 16)
    out = _tc_post(t2[0], t2[1], m2, dinv, b2.reshape(1, H), batch3d,
                   W_ih.T, b_ih.reshape(1, 4 * H), b_hh.reshape(1, 4 * H),
                   wout_t, bout)
    return out[:, :OUT]


# speedup vs baseline: 1.1106x; 1.1106x over previous
import functools

import jax
import jax.numpy as jnp
from jax import lax
from jax.experimental import pallas as pl
from jax.experimental.pallas import tpu as pltpu
from jax.experimental.pallas import tpu_sc as plsc

N = 10000
E = 320000
D = 128
H = 128
G = 64
OUT = 10

NPAD = 10240
RPT = NPAD // 16
CH = 128
CHUNKS = 2560
E_PAD = CHUNKS * CH
CPW = CHUNKS // 32
SLAB = 40
BN = 1024
NB = NPAD // BN

_mesh = plsc.VectorSubcoreMesh(core_axis_name="c", subcore_axis_name="s")


def _sc_deg_body(dst_hbm, ones_hbm, out_hbm, dstv, onesv, acc, ss0, ss1):
    c = lax.axis_index("c")
    s = lax.axis_index("s")
    r0 = s * RPT
    pltpu.sync_copy(ones_hbm, onesv)

    @pl.loop(0, RPT // CH)
    def _(k):
        pltpu.sync_copy(onesv, acc.at[pl.ds(r0 + k * CH, CH)])

    base = (c * 16 + s) * CPW
    pltpu.sync_copy(dst_hbm.at[pl.ds(base, CPW)], dstv)
    plsc.subcore_barrier()

    pltpu.async_copy(onesv, acc.at[dstv.at[0]], ss0, add=True)
    pltpu.async_copy(onesv, acc.at[dstv.at[1]], ss1, add=True)

    @pl.loop(0, CPW, step=2)
    def _(j):
        pltpu.make_async_copy(onesv, acc.at[dstv.at[j]], ss0).wait()

        @pl.when(j + 2 < CPW)
        def _():
            pltpu.async_copy(onesv, acc.at[dstv.at[j + 2]], ss0, add=True)

        pltpu.make_async_copy(onesv, acc.at[dstv.at[j + 1]], ss1).wait()

        @pl.when(j + 3 < CPW)
        def _():
            pltpu.async_copy(onesv, acc.at[dstv.at[j + 3]], ss1, add=True)

    plsc.subcore_barrier()
    pltpu.sync_copy(acc.at[pl.ds(r0, RPT)], out_hbm.at[c, pl.ds(r0, RPT)])


def _sc_agg_body(src_hbm, dst_hbm, m_hbm, out_hbm, srcv, dstv, rows0, rows1,
                 acc, gs0, gs1, ss0, ss1):
    c = lax.axis_index("c")
    s = lax.axis_index("s")
    r0 = s * RPT
    pltpu.sync_copy(m_hbm.at[pl.ds(r0, RPT)], acc.at[pl.ds(r0, RPT)])
    base = (c * 16 + s) * CPW
    plsc.subcore_barrier()

    for p in range(CPW // SLAB):
        pltpu.sync_copy(src_hbm.at[pl.ds(base + p * SLAB, SLAB)], srcv)
        pltpu.sync_copy(dst_hbm.at[pl.ds(base + p * SLAB, SLAB)], dstv)
        pltpu.async_copy(m_hbm.at[srcv.at[0]], rows0, gs0)

        @pl.loop(0, SLAB, step=2)
        def _(j):
            pltpu.async_copy(m_hbm.at[srcv.at[j + 1]], rows1, gs1)
            pltpu.make_async_copy(m_hbm.at[srcv.at[j]], rows0, gs0).wait()
            pltpu.sync_copy(rows0, acc.at[dstv.at[j]], add=True)

            @pl.when(j + 2 < SLAB)
            def _():
                pltpu.async_copy(m_hbm.at[srcv.at[j + 2]], rows0, gs0)

            pltpu.make_async_copy(m_hbm.at[srcv.at[j + 1]], rows1, gs1).wait()
            pltpu.sync_copy(rows1, acc.at[dstv.at[j + 1]], add=True)

    plsc.subcore_barrier()
    pltpu.sync_copy(acc.at[pl.ds(r0, RPT)], out_hbm.at[c, pl.ds(r0, RPT)])


_sc_deg = pl.kernel(
    _sc_deg_body,
    out_type=jax.ShapeDtypeStruct((2, NPAD, H), jnp.float32),
    mesh=_mesh,
    scratch_types=[
        pltpu.VMEM((CPW, CH), jnp.int32),
        pltpu.VMEM((CH, H), jnp.float32),
        pltpu.VMEM_SHARED((NPAD, H), jnp.float32),
        pltpu.SemaphoreType.DMA,
        pltpu.SemaphoreType.DMA,
    ],
)

_sc_agg = pl.kernel(
    _sc_agg_body,
    out_type=jax.ShapeDtypeStruct((2, NPAD, H), jnp.float32),
    mesh=_mesh,
    scratch_types=[
        pltpu.VMEM((SLAB, CH), jnp.int32),
        pltpu.VMEM((SLAB, CH), jnp.int32),
        pltpu.VMEM((CH, H), jnp.float32),
        pltpu.VMEM((CH, H), jnp.float32),
        pltpu.VMEM_SHARED((NPAD, H), jnp.float32),
        pltpu.SemaphoreType.DMA,
        pltpu.SemaphoreType.DMA,
        pltpu.SemaphoreType.DMA,
        pltpu.SemaphoreType.DMA,
    ],
)


def _tc_mm_body(x, w, u_out):
    u_out[...] = jnp.dot(x[...], w[...], preferred_element_type=jnp.float32)


def _tc_pre_body(dega, degb, u, m_out, dinv_out):
    i = pl.program_id(0)
    deg = dega[...][0][:, 0:1] + degb[...][0][:, 0:1] - 1.0
    rows = i * BN + lax.broadcasted_iota(jnp.int32, (BN, 1), 0)
    dinv = jnp.where(rows < N, lax.rsqrt(jnp.maximum(deg, 1e-12)), 0.0)
    m_out[...] = u[...] * dinv
    dinv_out[...] = dinv


def _tc_mid_body(ta, tb, m, dinv, b, w, out):
    t = ta[...][0] + tb[...][0] - m[...]
    h = jnp.maximum(dinv[...] * t + b[...], 0.0)
    out[...] = jnp.dot(h, w[...], preferred_element_type=jnp.float32) * dinv[...]


def _tc_post_body(ta, tb, m, dinv, b, bt, wih_t, bih, bhh, wout_t, bout,
                  out, sums, cnt):
    i = pl.program_id(0)

    @pl.when(i == 0)
    def _():
        sums[...] = jnp.zeros((G, H), jnp.float32)
        cnt[...] = jnp.zeros((G, 1), jnp.float32)

    t = ta[...][0] + tb[...][0] - m[...]
    h = jnp.maximum(dinv[...] * t + b[...], 0.0)
    bvec = bt[...][0]
    gids = lax.broadcasted_iota(jnp.int32, (G, 1), 0)
    p = (bvec == gids).astype(jnp.float32)
    sums[...] += jnp.dot(p, h, preferred_element_type=jnp.float32)
    cnt[...] += jnp.sum(p, axis=1, keepdims=True)

    @pl.when(i == NB - 1)
    def _():
        pooled = sums[...] / jnp.maximum(cnt[...], 1.0)
        gates = (jnp.dot(pooled, wih_t[...], preferred_element_type=jnp.float32)
                 + bih[...] + bhh[...])
        ig = jax.nn.sigmoid(gates[:, 0:H])
        gg = jnp.tanh(gates[:, 2 * H:3 * H])
        og = jax.nn.sigmoid(gates[:, 3 * H:4 * H])
        hn = og * jnp.tanh(ig * gg)
        out[...] = (jnp.dot(hn, wout_t[...], preferred_element_type=jnp.float32)
                    + bout[...])


def _row_spec(width):
    return pl.BlockSpec((BN, width), lambda i: (i, 0))


def _full_spec(shape):
    return pl.BlockSpec(shape, lambda i: (0,) * len(shape))


_tc_mm = pl.pallas_call(
    _tc_mm_body,
    grid=(NB,),
    in_specs=[_row_spec(D), _full_spec((D, H))],
    out_specs=_row_spec(H),
    out_shape=jax.ShapeDtypeStruct((NPAD, H), jnp.float32),
)

def _part_spec(core):
    return pl.BlockSpec((1, BN, H), lambda i, c=core: (c, i, 0))


_tc_pre = pl.pallas_call(
    _tc_pre_body,
    grid=(NB,),
    in_specs=[_part_spec(0), _part_spec(1), _row_spec(H)],
    out_specs=[_row_spec(H), _row_spec(1)],
    out_shape=[jax.ShapeDtypeStruct((NPAD, H), jnp.float32),
               jax.ShapeDtypeStruct((NPAD, 1), jnp.float32)],
)

_tc_mid = pl.pallas_call(
    _tc_mid_body,
    grid=(NB,),
    in_specs=[_part_spec(0), _part_spec(1), _row_spec(H), _row_spec(1),
              _full_spec((1, H)), _full_spec((H, H))],
    out_specs=_row_spec(H),
    out_shape=jax.ShapeDtypeStruct((NPAD, H), jnp.float32),
)

_tc_post = pl.pallas_call(
    _tc_post_body,
    grid=(NB,),
    in_specs=[_part_spec(0), _part_spec(1), _row_spec(H), _row_spec(1),
              _full_spec((1, H)),
              pl.BlockSpec((1, 1, BN), lambda i: (i, 0, 0)),
              _full_spec((H, 4 * H)), _full_spec((1, 4 * H)),
              _full_spec((1, 4 * H)), _full_spec((H, 16)),
              _full_spec((1, 16))],
    out_specs=_full_spec((G, 16)),
    out_shape=jax.ShapeDtypeStruct((G, 16), jnp.float32),
    scratch_shapes=[pltpu.VMEM((G, H), jnp.float32),
                    pltpu.VMEM((G, 1), jnp.float32)],
)


def kernel(x, edge_index, batch, W1, b1, W2, b2, W_ih, W_hh, b_ih, b_hh, W_out, b_out):
    src = edge_index[0]
    dst = edge_index[1]
    pad_e = E_PAD - E
    ar = jnp.arange(pad_e, dtype=jnp.int32)
    src2d = jnp.concatenate([src, ar % N]).reshape(CHUNKS, CH)
    dst2d = jnp.concatenate([dst, N + ar % (NPAD - N)]).reshape(CHUNKS, CH)
    x_pad = jnp.pad(x, ((0, NPAD - N), (0, 0)))
    batch3d = jnp.pad(batch, (0, NPAD - N), constant_values=G).reshape(NB, 1, BN)
    ones128 = jnp.ones((CH, H), jnp.float32)
    u1 = _tc_mm(x_pad, W1)
    degp = _sc_deg(dst2d, ones128)
    m1, dinv = _tc_pre(degp, degp, u1)
    t1 = _sc_agg(src2d, dst2d, m1)
    m2 = _tc_mid(t1, t1, m1, dinv, b1.reshape(1, H), W2)
    t2 = _sc_agg(src2d, dst2d, m2)
    wout_t = jnp.pad(W_out.T, ((0, 0), (0, 16 - OUT)))
    bout = jnp.pad(b_out, (0, 16 - OUT)).reshape(1, 16)
    out = _tc_post(t2, t2, m2, dinv, b2.reshape(1, H), batch3d,
                   W_ih.T, b_ih.reshape(1, 4 * H), b_hh.reshape(1, 4 * H),
                   wout_t, bout)
    return out[:, :OUT]


# speedup vs baseline: 1.1274x; 1.0151x over previous
import functools

import jax
import jax.numpy as jnp
from jax import lax
from jax.experimental import pallas as pl
from jax.experimental.pallas import tpu as pltpu
from jax.experimental.pallas import tpu_sc as plsc

N = 10000
E = 320000
D = 128
H = 128
G = 64
OUT = 10

NPAD = 10240
RPT = NPAD // 16
CH = 128
CHUNKS = 2560
E_PAD = CHUNKS * CH
CPW = CHUNKS // 32
SLAB = 40
BN = 1024
NB = NPAD // BN

_mesh = plsc.VectorSubcoreMesh(core_axis_name="c", subcore_axis_name="s")


def _sc_deg_body(edges_hbm, ones_hbm, out_hbm, dstv, onesv, acc, ss0, ss1):
    c = lax.axis_index("c")
    s = lax.axis_index("s")
    r0 = s * RPT
    pltpu.sync_copy(ones_hbm, onesv)

    @pl.loop(0, RPT // CH)
    def _(k):
        pltpu.sync_copy(onesv, acc.at[pl.ds(r0 + k * CH, CH)])

    base = (c * 16 + s) * CPW
    pltpu.sync_copy(edges_hbm.at[1, pl.ds(base, CPW)], dstv)
    plsc.subcore_barrier()

    pltpu.async_copy(onesv, acc.at[dstv.at[0]], ss0, add=True)
    pltpu.async_copy(onesv, acc.at[dstv.at[1]], ss1, add=True)

    @pl.loop(0, CPW, step=2)
    def _(j):
        pltpu.make_async_copy(onesv, acc.at[dstv.at[j]], ss0).wait()

        @pl.when(j + 2 < CPW)
        def _():
            pltpu.async_copy(onesv, acc.at[dstv.at[j + 2]], ss0, add=True)

        pltpu.make_async_copy(onesv, acc.at[dstv.at[j + 1]], ss1).wait()

        @pl.when(j + 3 < CPW)
        def _():
            pltpu.async_copy(onesv, acc.at[dstv.at[j + 3]], ss1, add=True)

    plsc.subcore_barrier()
    pltpu.sync_copy(acc.at[pl.ds(r0, RPT)], out_hbm.at[c, pl.ds(r0, RPT)])


def _sc_agg_body(edges_hbm, m_hbm, out_hbm, srcv, dstv, rows0, rows1,
                 acc, gs0, gs1, ss0, ss1):
    c = lax.axis_index("c")
    s = lax.axis_index("s")
    r0 = s * RPT
    pltpu.sync_copy(m_hbm.at[pl.ds(r0, RPT)], acc.at[pl.ds(r0, RPT)])
    base = (c * 16 + s) * CPW
    plsc.subcore_barrier()

    for p in range(CPW // SLAB):
        pltpu.sync_copy(edges_hbm.at[0, pl.ds(base + p * SLAB, SLAB)], srcv)
        pltpu.sync_copy(edges_hbm.at[1, pl.ds(base + p * SLAB, SLAB)], dstv)
        pltpu.async_copy(m_hbm.at[srcv.at[0]], rows0, gs0)

        @pl.loop(0, SLAB, step=2)
        def _(j):
            pltpu.async_copy(m_hbm.at[srcv.at[j + 1]], rows1, gs1)
            pltpu.make_async_copy(m_hbm.at[srcv.at[j]], rows0, gs0).wait()
            pltpu.sync_copy(rows0, acc.at[dstv.at[j]], add=True)

            @pl.when(j + 2 < SLAB)
            def _():
                pltpu.async_copy(m_hbm.at[srcv.at[j + 2]], rows0, gs0)

            pltpu.make_async_copy(m_hbm.at[srcv.at[j + 1]], rows1, gs1).wait()
            pltpu.sync_copy(rows1, acc.at[dstv.at[j + 1]], add=True)

    plsc.subcore_barrier()
    pltpu.sync_copy(acc.at[pl.ds(r0, RPT)], out_hbm.at[c, pl.ds(r0, RPT)])


_sc_deg = pl.kernel(
    _sc_deg_body,
    out_type=jax.ShapeDtypeStruct((2, NPAD, H), jnp.float32),
    mesh=_mesh,
    scratch_types=[
        pltpu.VMEM((CPW, CH), jnp.int32),
        pltpu.VMEM((CH, H), jnp.float32),
        pltpu.VMEM_SHARED((NPAD, H), jnp.float32),
        pltpu.SemaphoreType.DMA,
        pltpu.SemaphoreType.DMA,
    ],
)

_sc_agg = pl.kernel(
    _sc_agg_body,
    out_type=jax.ShapeDtypeStruct((2, NPAD, H), jnp.float32),
    mesh=_mesh,
    scratch_types=[
        pltpu.VMEM((SLAB, CH), jnp.int32),
        pltpu.VMEM((SLAB, CH), jnp.int32),
        pltpu.VMEM((CH, H), jnp.float32),
        pltpu.VMEM((CH, H), jnp.float32),
        pltpu.VMEM_SHARED((NPAD, H), jnp.float32),
        pltpu.SemaphoreType.DMA,
        pltpu.SemaphoreType.DMA,
        pltpu.SemaphoreType.DMA,
        pltpu.SemaphoreType.DMA,
    ],
)


def _tc_mm_body(x, w, u_out):
    u_out[...] = jnp.dot(x[...], w[...], preferred_element_type=jnp.float32)


def _tc_pre_body(dega, degb, u, m_out, dinv_out):
    i = pl.program_id(0)
    deg = dega[...][0][:, 0:1] + degb[...][0][:, 0:1] - 1.0
    rows = i * BN + lax.broadcasted_iota(jnp.int32, (BN, 1), 0)
    dinv = jnp.where(rows < N, lax.rsqrt(jnp.maximum(deg, 1e-12)), 0.0)
    m_out[...] = u[...] * dinv
    dinv_out[...] = dinv


def _tc_mid_body(ta, tb, m, dinv, b, w, out):
    t = ta[...][0] + tb[...][0] - m[...]
    h = jnp.maximum(dinv[...] * t + b[...], 0.0)
    out[...] = jnp.dot(h, w[...], preferred_element_type=jnp.float32) * dinv[...]


def _tc_post_body(ta, tb, m, dinv, b, bt, wih_t, bih, bhh, wout_t, bout,
                  out, sums, cnt):
    i = pl.program_id(0)

    @pl.when(i == 0)
    def _():
        sums[...] = jnp.zeros((G, H), jnp.float32)
        cnt[...] = jnp.zeros((G, 1), jnp.float32)

    t = ta[...][0] + tb[...][0] - m[...]
    h = jnp.maximum(dinv[...] * t + b[...], 0.0)
    bvec = bt[...][0]
    gids = lax.broadcasted_iota(jnp.int32, (G, 1), 0)
    p = (bvec == gids).astype(jnp.float32)
    sums[...] += jnp.dot(p, h, preferred_element_type=jnp.float32)
    cnt[...] += jnp.sum(p, axis=1, keepdims=True)

    @pl.when(i == NB - 1)
    def _():
        pooled = sums[...] / jnp.maximum(cnt[...], 1.0)
        gates = (jnp.dot(pooled, wih_t[...], preferred_element_type=jnp.float32)
                 + bih[...] + bhh[...])
        ig = jax.nn.sigmoid(gates[:, 0:H])
        gg = jnp.tanh(gates[:, 2 * H:3 * H])
        og = jax.nn.sigmoid(gates[:, 3 * H:4 * H])
        hn = og * jnp.tanh(ig * gg)
        out[...] = (jnp.dot(hn, wout_t[...], preferred_element_type=jnp.float32)
                    + bout[...])


def _row_spec(width):
    return pl.BlockSpec((BN, width), lambda i: (i, 0))


def _full_spec(shape):
    return pl.BlockSpec(shape, lambda i: (0,) * len(shape))


_tc_mm = pl.pallas_call(
    _tc_mm_body,
    grid=(NB,),
    in_specs=[_row_spec(D), _full_spec((D, H))],
    out_specs=_row_spec(H),
    out_shape=jax.ShapeDtypeStruct((NPAD, H), jnp.float32),
)

def _part_spec(core):
    return pl.BlockSpec((1, BN, H), lambda i, c=core: (c, i, 0))


_tc_pre = pl.pallas_call(
    _tc_pre_body,
    grid=(NB,),
    in_specs=[_part_spec(0), _part_spec(1), _row_spec(H)],
    out_specs=[_row_spec(H), _row_spec(1)],
    out_shape=[jax.ShapeDtypeStruct((NPAD, H), jnp.float32),
               jax.ShapeDtypeStruct((NPAD, 1), jnp.float32)],
)

_tc_mid = pl.pallas_call(
    _tc_mid_body,
    grid=(NB,),
    in_specs=[_part_spec(0), _part_spec(1), _row_spec(H), _row_spec(1),
              _full_spec((1, H)), _full_spec((H, H))],
    out_specs=_row_spec(H),
    out_shape=jax.ShapeDtypeStruct((NPAD, H), jnp.float32),
)

_tc_post = pl.pallas_call(
    _tc_post_body,
    grid=(NB,),
    in_specs=[_part_spec(0), _part_spec(1), _row_spec(H), _row_spec(1),
              _full_spec((1, H)),
              pl.BlockSpec((1, 1, BN), lambda i: (i, 0, 0)),
              _full_spec((H, 4 * H)), _full_spec((1, 4 * H)),
              _full_spec((1, 4 * H)), _full_spec((H, 16)),
              _full_spec((1, 16))],
    out_specs=_full_spec((G, 16)),
    out_shape=jax.ShapeDtypeStruct((G, 16), jnp.float32),
    scratch_shapes=[pltpu.VMEM((G, H), jnp.float32),
                    pltpu.VMEM((G, 1), jnp.float32)],
)


def kernel(x, edge_index, batch, W1, b1, W2, b2, W_ih, W_hh, b_ih, b_hh, W_out, b_out):
    pad_e = E_PAD - E
    ar = jnp.arange(pad_e, dtype=jnp.int32)
    pad_edges = jnp.stack([ar % N, N + ar % (NPAD - N)])
    edges = jnp.concatenate([edge_index, pad_edges], axis=1).reshape(2, CHUNKS, CH)
    x_pad = jnp.pad(x, ((0, NPAD - N), (0, 0)))
    batch3d = jnp.pad(batch, (0, NPAD - N), constant_values=G).reshape(NB, 1, BN)
    ones128 = jnp.ones((CH, H), jnp.float32)
    u1 = _tc_mm(x_pad, W1)
    degp = _sc_deg(edges, ones128)
    m1, dinv = _tc_pre(degp, degp, u1)
    t1 = _sc_agg(edges, m1)
    m2 = _tc_mid(t1, t1, m1, dinv, b1.reshape(1, H), W2)
    t2 = _sc_agg(edges, m2)
    wout_t = jnp.pad(W_out.T, ((0, 0), (0, 16 - OUT)))
    bout = jnp.pad(b_out, (0, 16 - OUT)).reshape(1, 16)
    out = _tc_post(t2, t2, m2, dinv, b2.reshape(1, H), batch3d,
                   W_ih.T, b_ih.reshape(1, 4 * H), b_hh.reshape(1, 4 * H),
                   wout_t, bout)
    return out[:, :OUT]
